# Initial kernel scaffold; baseline (speedup 1.0000x reference)
import jax
import jax.numpy as jnp
from jax.experimental import pallas as pl


def kernel(env_vectors, atom_attr, W1, b1, W2, b2, env_index, edge_index):
    raise NotImplementedError("write your pallas kernel here")


# speedup vs baseline: 2.7350x; 2.7350x over previous
import functools

import jax
import jax.numpy as jnp
from jax import lax
from jax.experimental import pallas as pl
from jax.experimental.pallas import tpu as pltpu
from jax.experimental.pallas import tpu_sc as plsc

RS = 3.0
RC = 6.0

NC = 2
NS = 16
NW = NC * NS

F32 = jnp.float32
I32 = jnp.int32


def _mesh():
    return plsc.VectorSubcoreMesh(core_axis_name="c", subcore_axis_name="s",
                                  num_cores=NC, num_subcores=NS)


_SC_PARAMS = pltpu.CompilerParams(use_tc_tiling_on_sc=False)


def _make_gather_attr(RB, N, A):
    @functools.partial(
        pl.kernel,
        out_type=(
            jax.ShapeDtypeStruct((RB, 128, A), F32),
            jax.ShapeDtypeStruct((RB, 128, A), F32),
        ),
        mesh=_mesh(),
        compiler_params=_SC_PARAMS,
        scratch_types=[
            pltpu.VMEM((128,), I32),
            pltpu.VMEM((128,), I32),
            pltpu.VMEM((128, A), F32),
            pltpu.VMEM((128, A), F32),
            pltpu.SemaphoreType.DMA,
            pltpu.SemaphoreType.DMA,
        ],
    )
    def gather_attr(idx0_hbm, idx1_hbm, atom_hbm, out0, out1, i0v, i1v, r0, r1,
                    s0, s1):
        c = lax.axis_index("c")
        s = lax.axis_index("s")
        wid = s * NC + c
        nrows = (RB - wid + NW - 1) // NW

        def body(i, carry):
            r = wid + i * NW
            pltpu.sync_copy(idx0_hbm.at[r], i0v)
            pltpu.sync_copy(idx1_hbm.at[r], i1v)
            cp0 = pltpu.async_copy(atom_hbm.at[i0v], r0, s0)
            cp1 = pltpu.async_copy(atom_hbm.at[i1v], r1, s1)
            cp0.wait()
            cp1.wait()
            pltpu.sync_copy(r0, out0.at[r])
            pltpu.sync_copy(r1, out1.at[r])
            return carry

        lax.fori_loop(0, nrows, body, 0)

    return gather_attr


def _msg_body(env_ref, a0_ref, a1_ref, w1_ref, b1_ref, w2_ref, b2_ref,
              out_ref):
    env = env_ref[...]
    a0 = a0_ref[...]
    a1 = a1_ref[...]
    w1 = w1_ref[...]
    b1 = b1_ref[...]
    w2 = w2_ref[...]
    b2 = b2_ref[...]

    x = env[:, 0:1]
    y = env[:, 1:2]
    z = env[:, 2:3]
    r2 = x * x + y * y + z * z
    r = jnp.sqrt(r2)
    r_safe = jnp.maximum(r, 1e-6)
    inv = 1.0 / r_safe
    t = (r - RC) / (RS - RC)
    poly = t * t * t * (10.0 + t * (-15.0 + 6.0 * t)) + 1.0
    mid = inv * poly
    snorm = jnp.where(r < RS, inv, jnp.where(r < RC, mid, jnp.zeros_like(r)))

    pre = snorm * w1[0:1, :] + b1
    for d in range(4):
        pre = pre + a0[:, d:d + 1] * w1[1 + d:2 + d, :]
        pre = pre + a1[:, d:d + 1] * w1[5 + d:6 + d, :]
    h1 = jnp.tanh(pre)
    h2 = jnp.dot(h1, w2, preferred_element_type=F32) + b2

    col10 = lax.broadcasted_iota(I32, (10, 32), 1)
    row10 = lax.broadcasted_iota(I32, (10, 32), 0)
    rmat = jnp.where((col10 < 30) & (col10 // 3 == row10), 1.0, 0.0)
    col3 = lax.broadcasted_iota(I32, (3, 32), 1)
    row3 = lax.broadcasted_iota(I32, (3, 32), 0)
    cmat = jnp.where((col3 < 30) & (col3 % 3 == row3), 1.0, 0.0)
    B = env.shape[0]
    cnt_col = jnp.where(lax.broadcasted_iota(I32, (B, 32), 1) == 30, 1.0, 0.0)
    out = (jnp.dot(h2, rmat, preferred_element_type=F32) *
           jnp.dot(env, cmat, preferred_element_type=F32)) + cnt_col
    out_ref[...] = out


def _run_msg(env_vectors, attr0, attr1, W1, b1, W2, b2, E, BE):
    grid = (E // BE,)
    return pl.pallas_call(
        _msg_body,
        grid=grid,
        in_specs=[
            pl.BlockSpec((BE, 3), lambda i: (i, 0)),
            pl.BlockSpec((BE, 4), lambda i: (i, 0)),
            pl.BlockSpec((BE, 4), lambda i: (i, 0)),
            pl.BlockSpec((9, 20), lambda i: (0, 0)),
            pl.BlockSpec((1, 20), lambda i: (0, 0)),
            pl.BlockSpec((20, 10), lambda i: (0, 0)),
            pl.BlockSpec((1, 10), lambda i: (0, 0)),
        ],
        out_specs=pl.BlockSpec((BE, 32), lambda i: (i, 0)),
        out_shape=jax.ShapeDtypeStruct((E, 32), F32),
    )(env_vectors, attr0, attr1, W1, b1.reshape(1, 20), W2, b2.reshape(1, 10))


def _make_scatter_msg(RB, N):
    RB_SC = RB // NC
    CH = 5
    NCHUNK = RB_SC // CH
    ZR = 125
    NROW_T = N // NS
    NZ = NROW_T // ZR

    @functools.partial(
        pl.kernel,
        out_type=jax.ShapeDtypeStruct((NC, N, 32), F32),
        mesh=_mesh(),
        compiler_params=_SC_PARAMS,
        scratch_types=[
            pltpu.VMEM_SHARED((N, 32), F32),
            pltpu.VMEM((CH, 128, 32), F32),
            pltpu.VMEM((CH, 128), I32),
            pltpu.VMEM((ZR, 32), F32),
        ],
    )
    def scatter_msg(msg_hbm, dst_hbm, out, accum, mbuf, idxbuf, zbuf):
        c = lax.axis_index("c")
        s = lax.axis_index("s")

        zero16 = jnp.zeros((16,), F32)

        def zrow(i, carry):
            zbuf[i, pl.ds(0, 16)] = zero16
            zbuf[i, pl.ds(16, 16)] = zero16
            return carry

        lax.fori_loop(0, ZR, zrow, 0)

        base = s * NROW_T

        def zcopy(k, carry):
            pltpu.sync_copy(zbuf, accum.at[pl.ds(base + k * ZR, ZR)])
            return carry

        lax.fori_loop(0, NZ, zcopy, 0)
        plsc.subcore_barrier()

        nch = (NCHUNK - s + NS - 1) // NS

        def sbody(k, carry):
            j = s + k * NS
            row0 = c * RB_SC + j * CH
            pltpu.sync_copy(msg_hbm.at[pl.ds(row0, CH)], mbuf)
            pltpu.sync_copy(dst_hbm.at[pl.ds(row0, CH)], idxbuf)
            for jj in range(CH):
                pltpu.sync_copy(mbuf.at[jj], accum.at[idxbuf.at[jj]],
                                add=True)
            return carry

        lax.fori_loop(0, nch, sbody, 0)
        plsc.subcore_barrier()

        def obody(k, carry):
            r0 = base + k * ZR
            pltpu.sync_copy(accum.at[pl.ds(r0, ZR)], out.at[c, pl.ds(r0, ZR)])
            return carry

        lax.fori_loop(0, NZ, obody, 0)

    return scatter_msg


def _gram_body(p_ref, out_ref, pad_ref):
    p = p_ref[...]
    sfull = p[0] + p[1]
    cnt = jnp.maximum(sfull[:, 30:31], 1.0)
    a = sfull[:, :30] / cnt

    j30 = lax.broadcasted_iota(I32, (30, 100), 0)
    m = lax.broadcasted_iota(I32, (30, 100), 1)
    out = None
    for c in range(3):
        m1 = jnp.where(j30 == 3 * (m // 10) + c, 1.0, 0.0)
        m2 = jnp.where(j30 == 3 * (m % 10) + c, 1.0, 0.0)
        term = (jnp.dot(a, m1, preferred_element_type=F32) *
                jnp.dot(a, m2, preferred_element_type=F32))
        out = term if out is None else out + term
    out_ref[...] = out
    B = out.shape[0]
    pad_ref[...] = jnp.concatenate([out, jnp.zeros((B, 12), F32)], axis=1)


def _run_gram(partials, N, BN):
    grid = (N // BN,)
    return pl.pallas_call(
        _gram_body,
        grid=grid,
        in_specs=[pl.BlockSpec((2, BN, 32), lambda i: (0, i, 0))],
        out_specs=(
            pl.BlockSpec((BN, 100), lambda i: (i, 0)),
            pl.BlockSpec((BN, 112), lambda i: (i, 0)),
        ),
        out_shape=(
            jax.ShapeDtypeStruct((N, 100), F32),
            jax.ShapeDtypeStruct((N, 112), F32),
        ),
    )(partials)


def _make_edge_gather(RB, N, P):
    @functools.partial(
        pl.kernel,
        out_type=jax.ShapeDtypeStruct((RB, 128, P), F32),
        mesh=_mesh(),
        compiler_params=_SC_PARAMS,
        scratch_types=[
            pltpu.VMEM((128,), I32),
            pltpu.VMEM((128,), I32),
            pltpu.VMEM((128, P), F32),
            pltpu.VMEM((128, P), F32),
            pltpu.SemaphoreType.DMA,
            pltpu.SemaphoreType.DMA,
        ],
    )
    def edge_gather(nd_hbm, idx0_hbm, idx1_hbm, out, i0v, i1v, r0, r1, s0, s1):
        c = lax.axis_index("c")
        s = lax.axis_index("s")
        wid = s * NC + c
        nrows = (RB - wid + NW - 1) // NW

        def body(i, carry):
            r = wid + i * NW
            pltpu.sync_copy(idx0_hbm.at[r], i0v)
            pltpu.sync_copy(idx1_hbm.at[r], i1v)
            cp0 = pltpu.async_copy(nd_hbm.at[i0v], r0, s0)
            cp1 = pltpu.async_copy(nd_hbm.at[i1v], r1, s1)
            cp0.wait()
            cp1.wait()

            def addrow(i2, carry2):
                for cc in range(P // 16):
                    sl = pl.ds(cc * 16, 16)
                    r0[i2, sl] = r0[i2, sl] + r1[i2, sl]
                return carry2

            lax.fori_loop(0, 128, addrow, 0)
            pltpu.sync_copy(r0, out.at[r])
            return carry

        lax.fori_loop(0, nrows, body, 0)

    return edge_gather


_DBG_STAGE1 = True
_DBG_STAGE2 = True
_DBG_STAGE3 = True
_DBG_STAGE4 = True
_DBG_STAGE5 = True


def kernel(env_vectors, atom_attr, W1, b1, W2, b2, env_index, edge_index):
    N = atom_attr.shape[0]
    E = env_vectors.shape[0]
    A = atom_attr.shape[1]
    RB = E // 128
    P = 112
    BE = 8000
    BN = 2000

    ei0 = env_index[0].reshape(RB, 128)
    ei1 = env_index[1].reshape(RB, 128)
    de0 = edge_index[0].reshape(RB, 128)
    de1 = edge_index[1].reshape(RB, 128)

    if _DBG_STAGE1:
        atom_pad = jnp.pad(atom_attr, ((0, 0), (0, 16 - A)))
        attr0_3d, attr1_3d = _make_gather_attr(RB, N, 16)(ei0, ei1, atom_pad)
        attr0 = attr0_3d.reshape(E, 16)[:, :A]
        attr1 = attr1_3d.reshape(E, 16)[:, :A]
    else:
        attr0 = jnp.take(atom_attr, env_index[0], axis=0)
        attr1 = jnp.take(atom_attr, env_index[1], axis=0)

    if _DBG_STAGE2:
        msg = _run_msg(env_vectors, attr0, attr1, W1, b1, W2, b2, E, BE)
    else:
        r = jnp.sqrt(jnp.sum(env_vectors * env_vectors, axis=1,
                             keepdims=True))
        r_safe = jnp.maximum(r, 1e-6)
        inv = 1.0 / r_safe
        t = (r - RC) / (RS - RC)
        poly = t * t * t * (10.0 + t * (-15.0 + 6.0 * t)) + 1.0
        snorm = jnp.where(r < RS, inv,
                          jnp.where(r < RC, inv * poly, jnp.zeros_like(r)))
        h = jnp.concatenate([snorm, attr0, attr1], axis=1)
        h1 = jnp.tanh(h @ W1 + b1)
        h2 = h1 @ W2 + b2
        outer = (h2[:, :, None] * env_vectors[:, None, :]).reshape(E, 30)
        msg = jnp.concatenate(
            [outer, jnp.ones((E, 1), F32), jnp.zeros((E, 1), F32)], axis=1)
    msg3d = msg.reshape(RB, 128, 32)

    if _DBG_STAGE3:
        partials = _make_scatter_msg(RB, N)(msg3d, ei1)
    else:
        acc = jax.ops.segment_sum(msg, env_index[1], num_segments=N)
        partials = jnp.stack([acc, jnp.zeros_like(acc)])

    if _DBG_STAGE4:
        node_desc, node_pad = _run_gram(partials, N, BN)
    else:
        sfull = partials[0] + partials[1]
        cnt = jnp.maximum(sfull[:, 30:31], 1.0)
        a3 = (sfull[:, :30] / cnt).reshape(N, 10, 3)
        gram = jnp.einsum('nij,nkj->nik', a3, a3)
        node_desc = gram.reshape(N, 100)
        node_pad = jnp.concatenate([node_desc, jnp.zeros((N, 12), F32)],
                                   axis=1)

    if _DBG_STAGE5:
        edge3d = _make_edge_gather(RB, N, P)(node_pad, de0, de1)
        edge_desc = edge3d.reshape(E, P)[:, :100]
    else:
        edge_desc = (jnp.take(node_desc, edge_index[0], axis=0) +
                     jnp.take(node_desc, edge_index[1], axis=0))

    return node_desc, edge_desc
